# baseline (speedup 1.0000x reference)
import functools

import jax
import jax.numpy as jnp
from jax import lax
from jax.experimental import pallas as pl
from jax.experimental.pallas import tpu as pltpu
from jax.experimental.pallas import tpu_sc as plsc

_B = 4096
_NB = 3 * _B
_L = 50
_V = 100001
_VP = 100352
_PR = 2048
_NC = 2
_NS = 16
_NW = _NC * _NS
_CB = 256
_NCHUNK = _NB // _CB


def _sc_pool(cols, idxc):
    mesh = plsc.VectorSubcoreMesh(core_axis_name="c", subcore_axis_name="s")

    @functools.partial(
        pl.kernel,
        mesh=mesh,
        out_type=jax.ShapeDtypeStruct((64, _NB), jnp.float32),
        scratch_types=[
            pltpu.VMEM((_VP,), jnp.int32),
            [pltpu.VMEM((_CB, _L), jnp.int32) for _ in range(2)],
            [pltpu.VMEM((2, _CB), jnp.float32) for _ in range(2)],
            pltpu.SemaphoreType.DMA,
            [pltpu.SemaphoreType.DMA for _ in range(2)],
            [pltpu.SemaphoreType.DMA for _ in range(2)],
        ],
        compiler_params=pltpu.CompilerParams(
            needs_layout_passes=False, use_tc_tiling_on_sc=False),
    )
    def k(cols_hbm, idxc_hbm, pooled_hbm, col_v, idxbufs, outbufs, csem,
          isems, osems):
        t = lax.axis_index("s") * _NC + lax.axis_index("c")
        pltpu.sync_copy(cols_hbm.at[t], col_v)

        def issue_idx(c, par):
            pltpu.async_copy(idxc_hbm.at[pl.ds(c * _CB, _CB), :],
                             idxbufs[par], isems[par])

        def drain_idx(c, par):
            pltpu.make_async_copy(idxc_hbm.at[pl.ds(c * _CB, _CB), :],
                                  idxbufs[par], isems[par]).wait()

        def wait_out(c, par):
            pltpu.make_async_copy(
                outbufs[par].at[0],
                pooled_hbm.at[t, pl.ds(c * _CB, _CB)], osems[par]).wait()
            pltpu.make_async_copy(
                outbufs[par].at[1],
                pooled_hbm.at[t + 32, pl.ds(c * _CB, _CB)], osems[par]).wait()

        def issue_out(c, par):
            pltpu.async_copy(
                outbufs[par].at[0],
                pooled_hbm.at[t, pl.ds(c * _CB, _CB)], osems[par])
            pltpu.async_copy(
                outbufs[par].at[1],
                pooled_hbm.at[t + 32, pl.ds(c * _CB, _CB)], osems[par])

        issue_idx(0, 0)
        issue_idx(1, 1)
        hmask = jnp.full((16,), -65536, jnp.int32)
        iota16 = lax.iota(jnp.int32, 16)

        def chunk_body(c, carry):
            def do(par):
                drain_idx(c, par)

                @pl.when(c >= 2)
                def _():
                    wait_out(c - 2, par)

                ib = idxbufs[par]
                for g in range(_CB // 16):
                    sl = pl.ds(g * 16, 16)
                    rows = iota16 + (g * 16)

                    def rb(r, accs):
                        al, ah = accs
                        i16 = plsc.load_gather(ib, [rows, jnp.full((16,), r, jnp.int32)])
                        w = plsc.load_gather(col_v, [i16])
                        al = al + plsc.bitcast(w << 16, jnp.float32)
                        ah = ah + plsc.bitcast(w & hmask, jnp.float32)
                        return al, ah

                    z = jnp.zeros((16,), jnp.float32)
                    al, ah = lax.fori_loop(0, _L, rb, (z, z), unroll=10)
                    outbufs[par][0, sl] = al
                    outbufs[par][1, sl] = ah
                issue_out(c, par)

                @pl.when(c + 2 < _NCHUNK)
                def _():
                    issue_idx(c + 2, par)

            @pl.when(lax.rem(c, 2) == 0)
            def _():
                do(0)

            @pl.when(lax.rem(c, 2) == 1)
            def _():
                do(1)

            return carry

        lax.fori_loop(0, _NCHUNK, chunk_body, 0)
        wait_out(_NCHUNK - 2, 0)
        wait_out(_NCHUNK - 1, 1)

    return k(cols, idxc)


def _tc_pack(table):
    def body(t_ref, o_ref):
        b = t_ref[...].astype(jnp.bfloat16)
        u = lax.bitcast_convert_type(b, jnp.uint16).astype(jnp.int32)
        w = (u[:, 32:64] << 16) | u[:, 0:32]
        o_ref[...] = w.T

    return pl.pallas_call(
        body,
        grid=(_VP // _PR,),
        in_specs=[pl.BlockSpec((_PR, 64), lambda i: (i, 0))],
        out_specs=pl.BlockSpec((32, _PR), lambda i: (0, i)),
        out_shape=jax.ShapeDtypeStruct((32, _VP), jnp.int32),
    )(table)


def _tc_loss(pooled, iq, ip, inn):
    def body(p_ref, iq_ref, ip_ref, in_ref, out_ref):
        q = p_ref[:, 0:_B]
        pp = p_ref[:, _B:2 * _B]
        nn = p_ref[:, 2 * _B:3 * _B]
        s1 = jnp.sum(q * pp, axis=0, keepdims=True)
        s2 = jnp.sum(q * nn, axis=0, keepdims=True)
        x = iq_ref[...] * (ip_ref[...] * s1 - in_ref[...] * s2)
        y = jnp.maximum(-x, 0.0) + jnp.log1p(jnp.exp(-jnp.abs(x)))
        out_ref[...] = (jnp.sum(y) / _B).reshape(1, 1)

    return pl.pallas_call(
        body,
        out_shape=jax.ShapeDtypeStruct((1, 1), jnp.float32),
    )(pooled, iq, ip, inn)


def kernel(query, pos_result, neg_result, query_len, pos_len, neg_len, emb_table):
    cols = _tc_pack(emb_table)

    idxc = jnp.concatenate(
        [query.astype(jnp.int32), pos_result.astype(jnp.int32),
         neg_result.astype(jnp.int32)], axis=0)

    def _inv(l):
        return (1.0 / jnp.maximum(l, 1).astype(jnp.float32)).reshape(1, _B)

    pooled = _sc_pool(cols, idxc)
    loss = _tc_loss(pooled, _inv(query_len), _inv(pos_len), _inv(neg_len))
    return loss[0, 0]


# speedup vs baseline: 1.2500x; 1.2500x over previous
import functools

import jax
import jax.numpy as jnp
from jax import lax
from jax.experimental import pallas as pl
from jax.experimental.pallas import tpu as pltpu
from jax.experimental.pallas import tpu_sc as plsc

_B = 4096
_NB = 3 * _B
_L = 50
_V = 100001
_VP = 100352
_PR = 2048
_NC = 2
_NS = 16
_NW = _NC * _NS
_CB = 256
_NCHUNK = _NB // _CB


def _sc_pool(cols, idxc):
    mesh = plsc.VectorSubcoreMesh(core_axis_name="c", subcore_axis_name="s")

    @functools.partial(
        pl.kernel,
        mesh=mesh,
        out_type=jax.ShapeDtypeStruct((64, _NB), jnp.float32),
        scratch_types=[
            pltpu.VMEM((_VP,), jnp.int32),
            [pltpu.VMEM((_L, _CB), jnp.int32) for _ in range(2)],
            [pltpu.VMEM((2, _CB), jnp.float32) for _ in range(2)],
            pltpu.SemaphoreType.DMA,
            [pltpu.SemaphoreType.DMA for _ in range(2)],
            [pltpu.SemaphoreType.DMA for _ in range(2)],
        ],
        compiler_params=pltpu.CompilerParams(
            needs_layout_passes=False, use_tc_tiling_on_sc=False),
    )
    def k(cols_hbm, idxc_hbm, pooled_hbm, col_v, idxbufs, outbufs, csem,
          isems, osems):
        t = lax.axis_index("s") * _NC + lax.axis_index("c")
        pltpu.sync_copy(cols_hbm.at[t], col_v)

        def issue_idx(c, par):
            pltpu.async_copy(idxc_hbm.at[c], idxbufs[par], isems[par])

        def drain_idx(c, par):
            pltpu.make_async_copy(
                idxc_hbm.at[c], idxbufs[par], isems[par]).wait()

        def wait_out(c, par):
            pltpu.make_async_copy(
                outbufs[par].at[0],
                pooled_hbm.at[t, pl.ds(c * _CB, _CB)], osems[par]).wait()
            pltpu.make_async_copy(
                outbufs[par].at[1],
                pooled_hbm.at[t + 32, pl.ds(c * _CB, _CB)], osems[par]).wait()

        def issue_out(c, par):
            pltpu.async_copy(
                outbufs[par].at[0],
                pooled_hbm.at[t, pl.ds(c * _CB, _CB)], osems[par])
            pltpu.async_copy(
                outbufs[par].at[1],
                pooled_hbm.at[t + 32, pl.ds(c * _CB, _CB)], osems[par])

        issue_idx(0, 0)
        issue_idx(1, 1)
        hmask = jnp.full((16,), -65536, jnp.int32)

        def chunk_body(c, carry):
            def do(par):
                drain_idx(c, par)

                @pl.when(c >= 2)
                def _():
                    wait_out(c - 2, par)

                ib = idxbufs[par]
                for g in range(_CB // 16):
                    sl = pl.ds(g * 16, 16)

                    def rb(r, accs):
                        al, ah = accs
                        w = plsc.load_gather(col_v, [ib[r, sl]])
                        al = al + plsc.bitcast(w << 16, jnp.float32)
                        ah = ah + plsc.bitcast(w & hmask, jnp.float32)
                        return al, ah

                    z = jnp.zeros((16,), jnp.float32)
                    al, ah = lax.fori_loop(0, _L, rb, (z, z), unroll=10)
                    outbufs[par][0, sl] = al
                    outbufs[par][1, sl] = ah
                issue_out(c, par)

                @pl.when(c + 2 < _NCHUNK)
                def _():
                    issue_idx(c + 2, par)

            @pl.when(lax.rem(c, 2) == 0)
            def _():
                do(0)

            @pl.when(lax.rem(c, 2) == 1)
            def _():
                do(1)

            return carry

        lax.fori_loop(0, _NCHUNK, chunk_body, 0)
        wait_out(_NCHUNK - 2, 0)
        wait_out(_NCHUNK - 1, 1)

    return k(cols, idxc)


def _tc_pack(table):
    def body(t_ref, o_ref):
        b = t_ref[...].astype(jnp.bfloat16)
        u = lax.bitcast_convert_type(b, jnp.uint16).astype(jnp.int32)
        w = (u[:, 32:64] << 16) | u[:, 0:32]
        o_ref[...] = w.T

    return pl.pallas_call(
        body,
        grid=(_VP // _PR,),
        in_specs=[pl.BlockSpec((_PR, 64), lambda i: (i, 0))],
        out_specs=pl.BlockSpec((32, _PR), lambda i: (0, i)),
        out_shape=jax.ShapeDtypeStruct((32, _VP), jnp.int32),
    )(table)


def _tc_loss(pooled, iq, ip, inn):
    def body(p_ref, iq_ref, ip_ref, in_ref, out_ref):
        q = p_ref[:, 0:_B]
        pp = p_ref[:, _B:2 * _B]
        nn = p_ref[:, 2 * _B:3 * _B]
        s1 = jnp.sum(q * pp, axis=0, keepdims=True)
        s2 = jnp.sum(q * nn, axis=0, keepdims=True)
        x = iq_ref[...] * (ip_ref[...] * s1 - in_ref[...] * s2)
        y = jnp.maximum(-x, 0.0) + jnp.log1p(jnp.exp(-jnp.abs(x)))
        out_ref[...] = (jnp.sum(y) / _B).reshape(1, 1)

    return pl.pallas_call(
        body,
        out_shape=jax.ShapeDtypeStruct((1, 1), jnp.float32),
    )(pooled, iq, ip, inn)


def kernel(query, pos_result, neg_result, query_len, pos_len, neg_len, emb_table):
    cols = _tc_pack(emb_table)

    idx = jnp.concatenate(
        [query.astype(jnp.int32), pos_result.astype(jnp.int32),
         neg_result.astype(jnp.int32)], axis=0)
    idxc = idx.T.reshape(_L, _NCHUNK, _CB).transpose(1, 0, 2)

    def _inv(l):
        return (1.0 / jnp.maximum(l, 1).astype(jnp.float32)).reshape(1, _B)

    pooled = _sc_pool(cols, idxc)
    loss = _tc_loss(pooled, _inv(query_len), _inv(pos_len), _inv(neg_len))
    return loss[0, 0]
